# baseline (speedup 1.0000x reference)
import functools

import jax
import jax.numpy as jnp
from jax import lax
from jax.experimental import pallas as pl
from jax.experimental.pallas import tpu as pltpu
from jax.experimental.pallas import tpu_sc as plsc

H = 1024
N = 4 * 8192
NC, NS, LANES = 2, 16, 16
NW = NC * NS
TPW = N // NW
T = 16
NCHUNK = TPW // T
VPT = H // LANES

OFF_EXP = 2
OFF_PCT = 2 + 256
OFF_FRAC = 2 + 256 + 102
NROWS = 2 + 256 + 102 + 1001


def _body(tab_h, pos_h, exp_h, pct_h, frac_h, d_h, out_h,
          pos_i, exp_i, pct_i, lo_i, hi_i, d_v,
          pos_b, exp_b, pct_b, lo_b, hi_b, sem):
  wid = lax.axis_index("s") * NC + lax.axis_index("c")
  wbase = wid * TPW

  pltpu.sync_copy(pos_h.at[pl.ds(wbase, TPW)], pos_i)
  pltpu.sync_copy(exp_h.at[pl.ds(wbase, TPW)], exp_i)
  pltpu.sync_copy(pct_h.at[pl.ds(wbase, TPW)], pct_i)
  pltpu.sync_copy(frac_h.at[pl.ds(wbase, TPW)], lo_i)
  pltpu.sync_copy(d_h.at[pl.ds(wbase, TPW)], d_v)

  def off_body(i, _):
    sl = pl.ds(i * LANES, LANES)
    exp_i[sl] = exp_i[sl] + OFF_EXP
    pct_i[sl] = pct_i[sl] + OFF_PCT
    f = lo_i[sl]
    lo_i[sl] = f + OFF_FRAC
    hi_i[sl] = f + (OFF_FRAC + 1)
    return 0

  lax.fori_loop(0, TPW // LANES, off_body, 0)

  def chunk_body(ci, _):
    cbase = ci * T
    g1 = pltpu.async_copy(tab_h.at[pos_i.at[pl.ds(cbase, T)]], pos_b, sem)
    g2 = pltpu.async_copy(tab_h.at[exp_i.at[pl.ds(cbase, T)]], exp_b, sem)
    g3 = pltpu.async_copy(tab_h.at[pct_i.at[pl.ds(cbase, T)]], pct_b, sem)
    g4 = pltpu.async_copy(tab_h.at[lo_i.at[pl.ds(cbase, T)]], lo_b, sem)
    g5 = pltpu.async_copy(tab_h.at[hi_i.at[pl.ds(cbase, T)]], hi_b, sem)
    g1.wait(); g2.wait(); g3.wait(); g4.wait(); g5.wait()

    d16 = d_v[pl.ds(cbase, LANES)]
    for t in range(T):
      dsp = jnp.broadcast_to(d16[t], (LANES,))
      one_m = 1.0 - dsp

      def vec_body(v, _, t=t, dsp=dsp, one_m=one_m):
        sl = pl.ds(v * LANES, LANES)
        acc = pos_b[t, sl] + exp_b[t, sl] + pct_b[t, sl]
        pos_b[t, sl] = acc + one_m * lo_b[t, sl] + dsp * hi_b[t, sl]
        return 0

      lax.fori_loop(0, VPT, vec_body, 0)
    pltpu.sync_copy(pos_b, out_h.at[pl.ds(wbase + cbase, T)])
    return 0

  lax.fori_loop(0, NCHUNK, chunk_body, 0)


def _run(tab, pos, exp, pct, frac, delta):
  mesh = plsc.VectorSubcoreMesh(core_axis_name="c", subcore_axis_name="s")
  fn = pl.kernel(
      _body,
      out_type=jax.ShapeDtypeStruct((N, H), jnp.float32),
      mesh=mesh,
      scratch_types=[
          pltpu.VMEM((TPW,), jnp.int32),
          pltpu.VMEM((TPW,), jnp.int32),
          pltpu.VMEM((TPW,), jnp.int32),
          pltpu.VMEM((TPW,), jnp.int32),
          pltpu.VMEM((TPW,), jnp.int32),
          pltpu.VMEM((TPW,), jnp.float32),
          pltpu.VMEM((T, H), jnp.float32),
          pltpu.VMEM((T, H), jnp.float32),
          pltpu.VMEM((T, H), jnp.float32),
          pltpu.VMEM((T, H), jnp.float32),
          pltpu.VMEM((T, H), jnp.float32),
          pltpu.SemaphoreType.DMA,
      ],
  )
  return fn(tab, pos, exp, pct, frac, delta)


def kernel(is_positive, exponent, fraction_bin, delta, percentile_values,
           W_pos, W_exp, W_frac, W_pct):
  B, L = is_positive.shape
  tab = jnp.concatenate([W_pos, W_exp, W_pct, W_frac], axis=0)
  pos = is_positive.astype(jnp.int32).reshape(N)
  exp = exponent.astype(jnp.int32).reshape(N)
  pct = percentile_values.astype(jnp.int32).reshape(N)
  frac = fraction_bin.astype(jnp.int32).reshape(N)
  d = delta.astype(jnp.float32).reshape(N)
  out = _run(tab, pos, exp, pct, frac, d)
  return out.reshape(B, L, H)


# speedup vs baseline: 1.9337x; 1.9337x over previous
import jax
import jax.numpy as jnp
from jax import lax
from jax.experimental import pallas as pl
from jax.experimental.pallas import tpu as pltpu
from jax.experimental.pallas import tpu_sc as plsc

H = 1024
HW = H // 2
N = 4 * 8192
NC, NS, LANES = 2, 16, 16
NW = NC * NS
TPW = N // NW
T = 16
G = 4 * T
NCHUNK = TPW // T
PAIRS = NCHUNK // 2

OFF_PCT = 512
OFF_FRAC = 512 + 102


def _body(tab_h, pos_h, exp_h, pct_h, frac_h, d_h, out_h, dum_h,
          pe_i, pct_i, fb_i, d_v, gidx,
          buf_a, buf_b, o_a, o_b, semg_a, semg_b, semo_a, semo_b):
  wid = lax.axis_index("s") * NC + lax.axis_index("c")
  wbase = wid * TPW

  pltpu.sync_copy(pos_h.at[pl.ds(wbase, TPW)], pe_i)
  pltpu.sync_copy(exp_h.at[pl.ds(wbase, TPW)], pct_i)
  pltpu.sync_copy(frac_h.at[pl.ds(wbase, TPW)], fb_i)
  pltpu.sync_copy(d_h.at[pl.ds(wbase, TPW)], d_v)

  def pe_body(i, _):
    sl = pl.ds(i * LANES, LANES)
    pe_i[sl] = pe_i[sl] * 256 + pct_i[sl]
    return 0

  lax.fori_loop(0, TPW // LANES, pe_body, 0)
  pltpu.sync_copy(pct_h.at[pl.ds(wbase, TPW)], pct_i)

  def gi_body(c, _):
    tb = c * T
    gidx[pl.ds(c * G, LANES)] = pe_i[pl.ds(tb, LANES)]
    gidx[pl.ds(c * G + T, LANES)] = pct_i[pl.ds(tb, LANES)] + OFF_PCT
    fb16 = fb_i[pl.ds(tb, LANES)]
    gidx[pl.ds(c * G + 2 * T, LANES)] = fb16 + OFF_FRAC
    gidx[pl.ds(c * G + 3 * T, LANES)] = fb16 + (OFF_FRAC + 1)
    return 0

  lax.fori_loop(0, NCHUNK, gi_body, 0)

  def fire_gather(c, buf, sem):
    return pltpu.async_copy(tab_h.at[gidx.at[pl.ds(c * G, G)]], buf, sem)

  def drain_gather(buf, sem):
    pltpu.make_async_copy(tab_h.at[gidx.at[pl.ds(0, G)]], buf, sem).wait()

  def fire_scatter(c, o, sem):
    return pltpu.async_copy(o, out_h.at[pl.ds((wbase + c * T) * H, T * H)], sem)

  def drain_scatter(o, sem):
    pltpu.make_async_copy(o, out_h.at[pl.ds(wbase * H, T * H)], sem).wait()

  fire_gather(0, buf_a, semg_a)
  fire_gather(1, buf_b, semg_b)
  pltpu.async_copy(o_a, dum_h.at[pl.ds(0, T * H)], semo_a)
  pltpu.async_copy(o_b, dum_h.at[pl.ds(T * H, T * H)], semo_b)

  lanes2 = lax.iota(jnp.int32, LANES) * 2

  def compute(c, buf, o):
    cb = c * T
    d16 = d_v[pl.ds(cb, LANES)]

    for t in range(T):
      dsp = jnp.broadcast_to(d16[t], (LANES,))
      one_m = 1.0 - dsp
      d32 = plsc.pack(dsp, dsp, format=plsc.PackFormat.INTERLEAVED)
      om32 = plsc.pack(one_m, one_m, format=plsc.PackFormat.INTERLEAVED)
      obase = t * H

      def vec_body(g, _, t=t, d32=d32, om32=om32, obase=obase):
        wbase_ = g * (8 * LANES)
        for u in range(8):
          sl = pl.ds(wbase_ + u * LANES, LANES)
          pe = plsc.bitcast(buf[t, sl], jnp.bfloat16)
          pc = plsc.bitcast(buf[T + t, sl], jnp.bfloat16)
          lo = plsc.bitcast(buf[2 * T + t, sl], jnp.bfloat16)
          hi = plsc.bitcast(buf[3 * T + t, sl], jnp.bfloat16)
          acc = pe + pc + om32 * lo + d32 * hi
          v = plsc.bitcast(acc, jnp.int32)
          ev = plsc.bitcast(lax.shift_left(v, 16), jnp.float32)
          od = plsc.bitcast(lax.bitwise_and(v, jnp.int32(-65536)), jnp.float32)
          epos = lanes2 + (obase + (wbase_ + u * LANES) * 2)
          plsc.store_scatter(o, [epos], ev)
          plsc.store_scatter(o, [epos + 1], od)
        return 0

      lax.fori_loop(0, HW // (8 * LANES), vec_body, 0)

  def pair_body(k, _):
    ca = 2 * k
    cb_ = 2 * k + 1
    drain_gather(buf_a, semg_a)
    drain_scatter(o_a, semo_a)
    compute(ca, buf_a, o_a)
    fire_scatter(ca, o_a, semo_a)
    fire_gather(jnp.minimum(ca + 2, NCHUNK - 2), buf_a, semg_a)
    drain_gather(buf_b, semg_b)
    drain_scatter(o_b, semo_b)
    compute(cb_, buf_b, o_b)
    fire_scatter(cb_, o_b, semo_b)
    fire_gather(jnp.minimum(cb_ + 2, NCHUNK - 1), buf_b, semg_b)
    return 0

  lax.fori_loop(0, PAIRS, pair_body, 0)

  drain_gather(buf_a, semg_a)
  drain_gather(buf_b, semg_b)
  drain_scatter(o_a, semo_a)
  drain_scatter(o_b, semo_b)


def _run(tab, pos, exp, pct, frac, delta):
  mesh = plsc.VectorSubcoreMesh(core_axis_name="c", subcore_axis_name="s")
  fn = pl.kernel(
      _body,
      out_type=(jax.ShapeDtypeStruct((N * H,), jnp.float32),
                jax.ShapeDtypeStruct((2 * T * H,), jnp.float32)),
      mesh=mesh,
      compiler_params=pltpu.CompilerParams(needs_layout_passes=False),
      scratch_types=[
          pltpu.VMEM((TPW,), jnp.int32),
          pltpu.VMEM((TPW,), jnp.int32),
          pltpu.VMEM((TPW,), jnp.int32),
          pltpu.VMEM((TPW,), jnp.float32),
          pltpu.VMEM((NCHUNK * G,), jnp.int32),
          pltpu.VMEM((G, HW), jnp.int32),
          pltpu.VMEM((G, HW), jnp.int32),
          pltpu.VMEM((T * H,), jnp.float32),
          pltpu.VMEM((T * H,), jnp.float32),
          pltpu.SemaphoreType.DMA,
          pltpu.SemaphoreType.DMA,
          pltpu.SemaphoreType.DMA,
          pltpu.SemaphoreType.DMA,
      ],
  )
  out, _ = fn(tab, pos, exp, pct, frac, delta)
  return out


def _pe_table(W_pos, W_exp):
  def body(p_ref, e_ref, o_ref):
    p = p_ref[...]
    e = e_ref[...]
    o_ref[...] = p[:, None, :] + e[None, :, :]

  return pl.pallas_call(
      body,
      out_shape=jax.ShapeDtypeStruct((2, 256, H), jnp.float32),
  )(W_pos, W_exp).reshape(512, H)


def kernel(is_positive, exponent, fraction_bin, delta, percentile_values,
           W_pos, W_exp, W_frac, W_pct):
  B, L = is_positive.shape
  tab = jnp.concatenate([_pe_table(W_pos, W_exp), W_pct, W_frac], axis=0)
  tab_p = lax.bitcast_convert_type(
      tab.astype(jnp.bfloat16).reshape(tab.shape[0], HW, 2), jnp.int32)
  pos = is_positive.astype(jnp.int32).reshape(N)
  exp = exponent.astype(jnp.int32).reshape(N)
  pct = percentile_values.astype(jnp.int32).reshape(N)
  frac = fraction_bin.astype(jnp.int32).reshape(N)
  d = delta.astype(jnp.float32).reshape(N)
  out = _run(tab_p, pos, exp, pct, frac, d)
  return out.reshape(B, L, H)


# speedup vs baseline: 6.0228x; 3.1147x over previous
import jax
import jax.numpy as jnp
from jax import lax
from jax.experimental import pallas as pl
from jax.experimental.pallas import tpu as pltpu
from jax.experimental.pallas import tpu_sc as plsc

H = 1024
HW = H // 2
N = 4 * 8192
NC, NS, LANES = 2, 16, 16
NW = NC * NS
TPW = N // NW
T = 8
G = 4 * T
NCHUNK = TPW // T
PAIRS = NCHUNK // 2
WPT = HW // LANES

OFF_PCT = 512
OFF_FRAC = 512 + 102


def _body(tab_h, pos_h, exp_h, pct_h, frac_h, d_h, out_h, dum_h,
          pe_i, pct_i, fb_i, d_v, gidx,
          buf_a, buf_b, o_a, o_b, semg_a, semg_b, semo_a, semo_b):
  wid = lax.axis_index("s") * NC + lax.axis_index("c")
  wbase = wid * TPW

  pltpu.sync_copy(pos_h.at[pl.ds(wbase, TPW)], pe_i)
  pltpu.sync_copy(exp_h.at[pl.ds(wbase, TPW)], pct_i)
  pltpu.sync_copy(frac_h.at[pl.ds(wbase, TPW)], fb_i)
  pltpu.sync_copy(d_h.at[pl.ds(wbase, TPW)], d_v.at[pl.ds(0, TPW)])

  def pe_body(i, _):
    sl = pl.ds(i * LANES, LANES)
    pe_i[sl] = pe_i[sl] * 256 + pct_i[sl]
    return 0

  lax.fori_loop(0, TPW // LANES, pe_body, 0)
  pltpu.sync_copy(pct_h.at[pl.ds(wbase, TPW)], pct_i)

  lanes = lax.iota(jnp.int32, LANES)
  dest0 = jnp.where(lanes < T, 0, G) + (lanes & (T - 1))

  def gi_body(p, _):
    tb = p * 2 * T
    dest = dest0 + p * (2 * G)
    pe16 = pe_i[pl.ds(tb, LANES)]
    pc16 = pct_i[pl.ds(tb, LANES)] + OFF_PCT
    fb16 = fb_i[pl.ds(tb, LANES)]
    plsc.store_scatter(gidx, [dest], pe16)
    plsc.store_scatter(gidx, [dest + T], pc16)
    plsc.store_scatter(gidx, [dest + 2 * T], fb16 + OFF_FRAC)
    plsc.store_scatter(gidx, [dest + 3 * T], fb16 + (OFF_FRAC + 1))
    return 0

  lax.fori_loop(0, PAIRS, gi_body, 0)

  def fire_gather(c, buf, sem):
    return pltpu.async_copy(tab_h.at[gidx.at[pl.ds(c * G, G)]], buf, sem)

  def drain_gather(buf, sem):
    pltpu.make_async_copy(tab_h.at[gidx.at[pl.ds(0, G)]], buf, sem).wait()

  def fire_scatter(c, o, sem):
    return pltpu.async_copy(o, out_h.at[pl.ds(wbase + c * T, T)], sem)

  def drain_scatter(o, sem):
    pltpu.make_async_copy(o, out_h.at[pl.ds(wbase, T)], sem).wait()

  fire_gather(0, buf_a, semg_a)
  fire_gather(1, buf_b, semg_b)
  pltpu.async_copy(o_a, dum_h.at[pl.ds(0, T)], semo_a)
  pltpu.async_copy(o_b, dum_h.at[pl.ds(T, T)], semo_b)

  def compute(c, buf, o):
    cb = c * T
    d16 = d_v[pl.ds(cb, LANES)]

    for t in range(T):
      dsp = jnp.broadcast_to(d16[t], (LANES,))
      one_m = 1.0 - dsp
      d32 = plsc.pack(dsp, dsp, format=plsc.PackFormat.INTERLEAVED)
      om32 = plsc.pack(one_m, one_m, format=plsc.PackFormat.INTERLEAVED)

      @plsc.parallel_loop(0, WPT, unroll=8)
      def _(w, t=t, d32=d32, om32=om32):
        slw = pl.ds(w * LANES, LANES)
        pe = plsc.bitcast(buf[t, slw], jnp.bfloat16)
        pc = plsc.bitcast(buf[T + t, slw], jnp.bfloat16)
        lo = plsc.bitcast(buf[2 * T + t, slw], jnp.bfloat16)
        hi = plsc.bitcast(buf[3 * T + t, slw], jnp.bfloat16)
        acc = pe + pc + om32 * lo + d32 * hi
        v = plsc.bitcast(acc, jnp.int32)
        ev = plsc.bitcast(lax.shift_left(v, 16), jnp.float32)
        od = plsc.bitcast(lax.bitwise_and(v, jnp.int32(-65536)), jnp.float32)
        col = w * (2 * LANES)
        o[t, pl.ds(col, LANES)] = ev
        o[t, pl.ds(col + LANES, LANES)] = od

  def pair_body(k, _):
    ca = 2 * k
    cb_ = 2 * k + 1
    drain_gather(buf_a, semg_a)
    drain_scatter(o_a, semo_a)
    compute(ca, buf_a, o_a)
    fire_scatter(ca, o_a, semo_a)
    fire_gather(jnp.minimum(ca + 2, NCHUNK - 2), buf_a, semg_a)
    drain_gather(buf_b, semg_b)
    drain_scatter(o_b, semo_b)
    compute(cb_, buf_b, o_b)
    fire_scatter(cb_, o_b, semo_b)
    fire_gather(jnp.minimum(cb_ + 2, NCHUNK - 1), buf_b, semg_b)
    return 0

  lax.fori_loop(0, PAIRS, pair_body, 0)

  drain_gather(buf_a, semg_a)
  drain_gather(buf_b, semg_b)
  drain_scatter(o_a, semo_a)
  drain_scatter(o_b, semo_b)


def _run(tab, pos, exp, pct, frac, delta):
  mesh = plsc.VectorSubcoreMesh(core_axis_name="c", subcore_axis_name="s")
  fn = pl.kernel(
      _body,
      out_type=(jax.ShapeDtypeStruct((N, H), jnp.float32),
                jax.ShapeDtypeStruct((2 * T, H), jnp.float32)),
      mesh=mesh,
      compiler_params=pltpu.CompilerParams(needs_layout_passes=False),
      scratch_types=[
          pltpu.VMEM((TPW,), jnp.int32),
          pltpu.VMEM((TPW,), jnp.int32),
          pltpu.VMEM((TPW,), jnp.int32),
          pltpu.VMEM((TPW + LANES,), jnp.float32),
          pltpu.VMEM((NCHUNK * G,), jnp.int32),
          pltpu.VMEM((G, HW), jnp.int32),
          pltpu.VMEM((G, HW), jnp.int32),
          pltpu.VMEM((T, H), jnp.float32),
          pltpu.VMEM((T, H), jnp.float32),
          pltpu.SemaphoreType.DMA,
          pltpu.SemaphoreType.DMA,
          pltpu.SemaphoreType.DMA,
          pltpu.SemaphoreType.DMA,
      ],
  )
  out, _ = fn(tab, pos, exp, pct, frac, delta)
  return out


def _pe_table(W_pos, W_exp):
  def body(p_ref, e_ref, o_ref):
    p = p_ref[...]
    e = e_ref[...]
    o_ref[...] = p[:, None, :] + e[None, :, :]

  return pl.pallas_call(
      body,
      out_shape=jax.ShapeDtypeStruct((2, 256, H), jnp.float32),
  )(W_pos, W_exp).reshape(512, H)


def kernel(is_positive, exponent, fraction_bin, delta, percentile_values,
           W_pos, W_exp, W_frac, W_pct):
  B, L = is_positive.shape
  tab = jnp.concatenate([_pe_table(W_pos, W_exp), W_pct, W_frac], axis=0)
  tab_b = tab.astype(jnp.bfloat16).reshape(tab.shape[0], H // 32, 2, LANES)
  tab_p = lax.bitcast_convert_type(tab_b.transpose(0, 1, 3, 2), jnp.int32)
  tab_p = tab_p.reshape(tab.shape[0], HW)
  pos = is_positive.astype(jnp.int32).reshape(N)
  exp = exponent.astype(jnp.int32).reshape(N)
  pct = percentile_values.astype(jnp.int32).reshape(N)
  frac = fraction_bin.astype(jnp.int32).reshape(N)
  d = delta.astype(jnp.float32).reshape(N)
  out = _run(tab_p, pos, exp, pct, frac, d)
  return out.reshape(B, L, H)
